# baseline (speedup 1.0000x reference)
import jax
import jax.numpy as jnp
from jax import lax
from jax.experimental import pallas as pl
from jax.experimental.pallas import tpu as pltpu
from jax.experimental.pallas import tpu_sc as plsc

_B = 16384
_D = 64
_NC = 2
_NS = 16
_NW = _NC * _NS
_BPW = _B // _NW
_CHUNK = 128
_NCHUNK = _BPW // _CHUNK
_GROUPS = _BPW // 16


def _body(ent_hbm, rel_hbm, heads_hbm, rels_hbm, tails_hbm, out_hbm,
          hidx, ridx, tidx, e1_v, r_v, e2_v, out_v, sem):
    wid = lax.axis_index("s") * _NC + lax.axis_index("c")
    base = wid * _BPW

    pltpu.sync_copy(heads_hbm.at[pl.ds(base, _BPW)], hidx)
    pltpu.sync_copy(rels_hbm.at[pl.ds(base, _BPW)], ridx)
    pltpu.sync_copy(tails_hbm.at[pl.ds(base, _BPW)], tidx)

    copies = []
    for k in range(_NCHUNK):
        sl = pl.ds(k * _CHUNK, _CHUNK)
        copies.append(pltpu.async_copy(ent_hbm.at[hidx.at[sl]], e1_v.at[sl], sem))
        copies.append(pltpu.async_copy(rel_hbm.at[ridx.at[sl]], r_v.at[sl], sem))
        copies.append(pltpu.async_copy(ent_hbm.at[tidx.at[sl]], e2_v.at[sl], sem))
    for c in copies:
        c.wait()

    lanes16 = lax.iota(jnp.int32, 16)
    bfly = [jnp.bitwise_xor(lanes16, sh) for sh in (8, 4, 2, 1)]

    dnums = lax.GatherDimensionNumbers(
        offset_dims=(), collapsed_slice_dims=(0,), start_index_map=(0,))

    def shuffle(v, idx):
        return lax.gather(v, idx[:, None], dnums, slice_sizes=(1,),
                          mode=lax.GatherScatterMode.PROMISE_IN_BOUNDS)

    def lanesum(v):
        for idx in bfly:
            v = v + shuffle(v, idx)
        return v

    def group(g, carry):
        row0 = g * 16
        s = jnp.zeros((16,), jnp.float32)
        for j in range(16):
            row = row0 + j
            acc = (e1_v[row, pl.ds(0, 16)] * r_v[row, pl.ds(0, 16)]) * e2_v[row, pl.ds(0, 16)]
            for c in range(1, _D // 16):
                acc = acc + (e1_v[row, pl.ds(c * 16, 16)] * r_v[row, pl.ds(c * 16, 16)]) \
                    * e2_v[row, pl.ds(c * 16, 16)]
            s = jnp.where(lanes16 == j, lanesum(acc), s)
        out_v[pl.ds(row0, 16)] = 1.0 / (1.0 + jnp.exp(-s))
        return carry

    lax.fori_loop(0, _GROUPS, group, 0)
    pltpu.sync_copy(out_v, out_hbm.at[pl.ds(base, _BPW)])


def kernel(entity_embedding, relation_embedding, heads, relations, tails):
    mesh = plsc.VectorSubcoreMesh(core_axis_name="c", subcore_axis_name="s")
    run = pl.kernel(
        _body,
        out_type=jax.ShapeDtypeStruct((_B,), jnp.float32),
        mesh=mesh,
        compiler_params=pltpu.CompilerParams(use_tc_tiling_on_sc=False),
        scratch_types=[
            pltpu.VMEM((_BPW,), jnp.int32),
            pltpu.VMEM((_BPW,), jnp.int32),
            pltpu.VMEM((_BPW,), jnp.int32),
            pltpu.VMEM((_BPW, _D), jnp.float32),
            pltpu.VMEM((_BPW, _D), jnp.float32),
            pltpu.VMEM((_BPW, _D), jnp.float32),
            pltpu.VMEM((_BPW,), jnp.float32),
            pltpu.SemaphoreType.DMA,
        ],
    )
    return run(entity_embedding, relation_embedding,
               heads.astype(jnp.int32), relations.astype(jnp.int32),
               tails.astype(jnp.int32))
